# baseline (speedup 1.0000x reference)
import jax
import jax.numpy as jnp
from jax.experimental import pallas as pl
from jax.experimental.pallas import tpu as pltpu

_N = 20000
_TM = 2000


def _elu(v):
    return jnp.where(v > 0, v, jnp.exp(v) - 1.0)


def _mega_body(x, wgt1, wgt2, wgt3, wgt4, wlt1, wlt2, wlt3, wlt4,
               bg1, bg2, bg3, bg4,
               pred, u1, u2, u3, u4,
               s0, s1, s2, s3, c1, c2, c3, c4):
    inv_n = 1.0 / _N
    steps = _N // _TM
    bf = jnp.bfloat16
    f32 = jnp.float32

    for s in (s0, s1, s2, s3):
        s[...] = jnp.zeros_like(s)

    def row_specs(k, o):
        return dict(in_specs=[pl.BlockSpec((_TM, k), lambda i: (i, 0))],
                    out_specs=[pl.BlockSpec((_TM, o), lambda i: (i, 0))])

    def head_body(x_v, u1_v):
        xb = x_v[...]
        u = jnp.dot(xb, wgt1[...], preferred_element_type=f32) + bg1[...]
        u1_v[...] = u.astype(bf)
        s0[...] += jnp.sum(xb.astype(f32), axis=0, keepdims=True)

    def mid_body(c_ref, wgt, bg, s_ref):
        def body(uin_v, uout_v):
            h = _elu(uin_v[...].astype(f32) - c_ref[...])
            u = jnp.dot(h.astype(bf), wgt[...],
                        preferred_element_type=f32) + bg[...]
            uout_v[...] = u.astype(uout_v.dtype)
            s_ref[...] += jnp.sum(h, axis=0, keepdims=True)
        return body

    def mean_row(s_ref, wlt):
        return jnp.dot((s_ref[...] * inv_n).astype(bf), wlt[...],
                       preferred_element_type=f32)

    pltpu.emit_pipeline(head_body, grid=(steps,),
                        **row_specs(1033, 1000))(x, u1)
    c1[...] = mean_row(s0, wlt1)

    pltpu.emit_pipeline(mid_body(c1, wgt2, bg2, s1), grid=(steps,),
                        **row_specs(1000, 600))(u1, u2)
    c2[...] = mean_row(s1, wlt2)

    pltpu.emit_pipeline(mid_body(c2, wgt3, bg3, s2), grid=(steps,),
                        **row_specs(600, 300))(u2, u3)
    c3[...] = mean_row(s2, wlt3)

    pltpu.emit_pipeline(mid_body(c3, wgt4, bg4, s3), grid=(steps,),
                        **row_specs(300, 1))(u3, u4)
    c4[...] = mean_row(s3, wlt4)

    def tail_body(u4_v, pred_v):
        pred_v[...] = _elu(u4_v[...] - c4[...])

    pltpu.emit_pipeline(tail_body, grid=(steps,),
                        **row_specs(1, 1))(u4, pred)


def kernel(x, Wg1, bg1, Wl1, Wg2, bg2, Wl2, Wg3, bg3, Wl3, Wg4, bg4, Wl4):
    bf = jnp.bfloat16
    f32 = jnp.float32
    wgt = [w.T.astype(bf) for w in (Wg1, Wg2, Wg3, Wg4)]
    wlt = [w.T.astype(bf) for w in (Wl1, Wl2, Wl3, Wl4)]
    bgs = [b.reshape(1, -1).astype(f32) for b in (bg1, bg2, bg3, bg4)]

    hbm = pl.BlockSpec(memory_space=pltpu.MemorySpace.HBM)
    vmem = pl.BlockSpec(memory_space=pltpu.MemorySpace.VMEM)

    outs = pl.pallas_call(
        _mega_body,
        in_specs=[hbm] + [vmem] * 12,
        out_specs=[hbm] * 5,
        out_shape=[
            jax.ShapeDtypeStruct((_N, 1), f32),
            jax.ShapeDtypeStruct((_N, 1000), bf),
            jax.ShapeDtypeStruct((_N, 600), bf),
            jax.ShapeDtypeStruct((_N, 300), bf),
            jax.ShapeDtypeStruct((_N, 1), f32),
        ],
        scratch_shapes=[
            pltpu.VMEM((1, 1033), f32),
            pltpu.VMEM((1, 1000), f32),
            pltpu.VMEM((1, 600), f32),
            pltpu.VMEM((1, 300), f32),
            pltpu.VMEM((1, 1000), f32),
            pltpu.VMEM((1, 600), f32),
            pltpu.VMEM((1, 300), f32),
            pltpu.VMEM((1, 1), f32),
        ],
    )(x.astype(bf), *wgt, *wlt, *bgs)
    return outs[0]


# speedup vs baseline: 1.1520x; 1.1520x over previous
import jax
import jax.numpy as jnp
from jax.experimental import pallas as pl
from jax.experimental.pallas import tpu as pltpu

_N = 20000
_NP = 20480
_TM = 2048
_PAD = _NP - _N


def _elu(v):
    return jnp.where(v > 0, v, jnp.exp(v) - 1.0)


def _mega_body(xt, wg1, wg2, wg3, wg4, wl1, wl2, wl3, wl4,
               bg1, bg2, bg3, bg4,
               pred, u1, u2, u3, u4,
               s0, s1, s2, s3, c1, c2, c3, c4):
    inv_n = 1.0 / _N
    steps = _NP // _TM
    bf = jnp.bfloat16
    f32 = jnp.float32

    for s in (s0, s1, s2, s3):
        s[...] = jnp.zeros_like(s)

    def col_specs(k, o):
        return dict(in_specs=[pl.BlockSpec((k, _TM), lambda i: (0, i))],
                    out_specs=[pl.BlockSpec((o, _TM), lambda i: (0, i))])

    def head_body(x_v, u1_v):
        xb = x_v[...]
        u = jnp.dot(wg1[...], xb, preferred_element_type=f32) + bg1[...]
        u1_v[...] = u.astype(bf)
        s0[...] += jnp.sum(xb.astype(f32), axis=1, keepdims=True)

    def mid_body(c_ref, wg, bg, s_ref):
        def body(uin_v, uout_v):
            h = _elu(uin_v[...].astype(f32) - c_ref[...])
            u = jnp.dot(wg[...], h.astype(bf),
                        preferred_element_type=f32) + bg[...]
            uout_v[...] = u.astype(uout_v.dtype)
            s_ref[...] += jnp.sum(h, axis=1, keepdims=True)
        return body

    def mean_col(wl, s_ref):
        return jnp.dot(wl[...], (s_ref[...] * inv_n).astype(bf),
                       preferred_element_type=f32)

    def matvec(w, v):
        return jnp.dot(w[...], v.astype(bf), preferred_element_type=f32)

    pltpu.emit_pipeline(head_body, grid=(steps,),
                        **col_specs(1033, 1000))(xt, u1)
    c1v = mean_col(wl1, s0)
    c1[...] = c1v
    hp1 = _elu(bg1[...].astype(bf).astype(f32) - c1v)

    pltpu.emit_pipeline(mid_body(c1, wg2, bg2, s1), grid=(steps,),
                        **col_specs(1000, 600))(u1, u2)
    s1[...] -= _PAD * hp1
    c2v = mean_col(wl2, s1)
    c2[...] = c2v
    hp2 = _elu((matvec(wg2, hp1) + bg2[...]).astype(bf).astype(f32) - c2v)

    pltpu.emit_pipeline(mid_body(c2, wg3, bg3, s2), grid=(steps,),
                        **col_specs(600, 300))(u2, u3)
    s2[...] -= _PAD * hp2
    c3v = mean_col(wl3, s2)
    c3[...] = c3v
    hp3 = _elu((matvec(wg3, hp2) + bg3[...]).astype(bf).astype(f32) - c3v)

    pltpu.emit_pipeline(mid_body(c3, wg4, bg4, s3), grid=(steps,),
                        **col_specs(300, 1))(u3, u4)
    s3[...] -= _PAD * hp3
    c4[...] = mean_col(wl4, s3)

    def tail_body(u4_v, pred_v):
        pred_v[...] = _elu(u4_v[...] - c4[...])

    pltpu.emit_pipeline(tail_body, grid=(steps,),
                        **col_specs(1, 1))(u4, pred)


def kernel(x, Wg1, bg1, Wl1, Wg2, bg2, Wl2, Wg3, bg3, Wl3, Wg4, bg4, Wl4):
    bf = jnp.bfloat16
    f32 = jnp.float32
    wg = [w.astype(bf) for w in (Wg1, Wg2, Wg3, Wg4)]
    wl = [w.astype(bf) for w in (Wl1, Wl2, Wl3, Wl4)]
    bgs = [b.reshape(-1, 1).astype(f32) for b in (bg1, bg2, bg3, bg4)]

    xt = jnp.pad(x.T.astype(bf), ((0, 0), (0, _PAD)))

    hbm = pl.BlockSpec(memory_space=pltpu.MemorySpace.HBM)
    vmem = pl.BlockSpec(memory_space=pltpu.MemorySpace.VMEM)

    outs = pl.pallas_call(
        _mega_body,
        in_specs=[hbm] + [vmem] * 12,
        out_specs=[hbm] * 5,
        out_shape=[
            jax.ShapeDtypeStruct((1, _NP), f32),
            jax.ShapeDtypeStruct((1000, _NP), bf),
            jax.ShapeDtypeStruct((600, _NP), bf),
            jax.ShapeDtypeStruct((300, _NP), bf),
            jax.ShapeDtypeStruct((1, _NP), f32),
        ],
        scratch_shapes=[
            pltpu.VMEM((1033, 1), f32),
            pltpu.VMEM((1000, 1), f32),
            pltpu.VMEM((600, 1), f32),
            pltpu.VMEM((300, 1), f32),
            pltpu.VMEM((1000, 1), f32),
            pltpu.VMEM((600, 1), f32),
            pltpu.VMEM((300, 1), f32),
            pltpu.VMEM((1, 1), f32),
        ],
    )(xt, *wg, *wl, *bgs)
    return outs[0][:, :_N].T


# speedup vs baseline: 1.4798x; 1.2845x over previous
import jax
import jax.numpy as jnp
from jax.experimental import pallas as pl
from jax.experimental.pallas import tpu as pltpu

_N = 20000
_TM = 1664
_STEPS = 12
_NM = _TM * _STEPS
_NR = _N - _NM


def _elu(v):
    return jnp.where(v > 0, v, jnp.exp(v) - 1.0)


def _mega_body(xt, xr, wg1, wg2, wg3, wg4, wl1, wl2, wl3, wl4,
               bg1, bg2, bg3, bg4,
               pred, u1, u2, u3, u4, predr,
               s0, s1, s2, s3, c1, c2, c3, c4, u1r, u2r, u3r, u4r):
    inv_n = 1.0 / _N
    bf = jnp.bfloat16
    f32 = jnp.float32

    for s in (s0, s1, s2, s3):
        s[...] = jnp.zeros_like(s)

    def col_specs(k, o):
        return dict(in_specs=[pl.BlockSpec((k, _TM), lambda i: (0, i))],
                    out_specs=[pl.BlockSpec((o, _TM), lambda i: (0, i))])

    def head_body(x_v, u1_v):
        xb = x_v[...]
        u = jnp.dot(wg1[...], xb.astype(bf),
                    preferred_element_type=f32) + bg1[...]
        u1_v[...] = u.astype(bf)
        s0[...] += jnp.sum(xb, axis=1, keepdims=True)

    def mid_body(c_ref, wg, bg, s_ref):
        def body(uin_v, uout_v):
            h = _elu(uin_v[...].astype(f32) - c_ref[...])
            u = jnp.dot(wg[...], h.astype(bf),
                        preferred_element_type=f32) + bg[...]
            uout_v[...] = u.astype(uout_v.dtype)
            s_ref[...] += jnp.sum(h, axis=1, keepdims=True)
        return body

    def mean_col(wl, s_ref):
        return jnp.dot(wl[...], (s_ref[...] * inv_n).astype(bf),
                       preferred_element_type=f32)

    def rem_step(uin_r, c_ref, wg, bg, s_ref, uout_r):
        h = _elu(uin_r[...].astype(f32) - c_ref[...])
        s_ref[...] += jnp.sum(h, axis=1, keepdims=True)
        u = jnp.dot(wg[...], h.astype(bf),
                    preferred_element_type=f32) + bg[...]
        uout_r[...] = u.astype(uout_r.dtype)

    pltpu.emit_pipeline(head_body, grid=(_STEPS,),
                        **col_specs(1033, 1000))(xt, u1)
    xrb = xr[...]
    s0[...] += jnp.sum(xrb, axis=1, keepdims=True)
    u1r[...] = (jnp.dot(wg1[...], xrb.astype(bf),
                        preferred_element_type=f32) + bg1[...]).astype(bf)
    c1[...] = mean_col(wl1, s0)

    pltpu.emit_pipeline(mid_body(c1, wg2, bg2, s1), grid=(_STEPS,),
                        **col_specs(1000, 600))(u1, u2)
    rem_step(u1r, c1, wg2, bg2, s1, u2r)
    c2[...] = mean_col(wl2, s1)

    pltpu.emit_pipeline(mid_body(c2, wg3, bg3, s2), grid=(_STEPS,),
                        **col_specs(600, 300))(u2, u3)
    rem_step(u2r, c2, wg3, bg3, s2, u3r)
    c3[...] = mean_col(wl3, s2)

    pltpu.emit_pipeline(mid_body(c3, wg4, bg4, s3), grid=(_STEPS,),
                        **col_specs(300, 1))(u3, u4)
    rem_step(u3r, c3, wg4, bg4, s3, u4r)
    c4[...] = mean_col(wl4, s3)

    def tail_body(u4_v, pred_v):
        pred_v[...] = _elu(u4_v[...] - c4[...])

    pltpu.emit_pipeline(tail_body, grid=(_STEPS,),
                        **col_specs(1, 1))(u4, pred)
    predr[...] = _elu(u4r[...] - c4[...])


def kernel(x, Wg1, bg1, Wl1, Wg2, bg2, Wl2, Wg3, bg3, Wl3, Wg4, bg4, Wl4):
    bf = jnp.bfloat16
    f32 = jnp.float32
    wg = [w.astype(bf) for w in (Wg1, Wg2, Wg3, Wg4)]
    wl = [w.astype(bf) for w in (Wl1, Wl2, Wl3, Wl4)]
    bgs = [b.reshape(-1, 1).astype(f32) for b in (bg1, bg2, bg3, bg4)]

    xt = x.T
    xr = x[_NM:, :].T

    hbm = pl.BlockSpec(memory_space=pltpu.MemorySpace.HBM)
    vmem = pl.BlockSpec(memory_space=pltpu.MemorySpace.VMEM)

    outs = pl.pallas_call(
        _mega_body,
        in_specs=[hbm] + [vmem] * 13,
        out_specs=[hbm] * 5 + [vmem],
        out_shape=[
            jax.ShapeDtypeStruct((1, _NM), f32),
            jax.ShapeDtypeStruct((1000, _NM), bf),
            jax.ShapeDtypeStruct((600, _NM), bf),
            jax.ShapeDtypeStruct((300, _NM), bf),
            jax.ShapeDtypeStruct((1, _NM), f32),
            jax.ShapeDtypeStruct((1, _NR), f32),
        ],
        scratch_shapes=[
            pltpu.VMEM((1033, 1), f32),
            pltpu.VMEM((1000, 1), f32),
            pltpu.VMEM((600, 1), f32),
            pltpu.VMEM((300, 1), f32),
            pltpu.VMEM((1000, 1), f32),
            pltpu.VMEM((600, 1), f32),
            pltpu.VMEM((300, 1), f32),
            pltpu.VMEM((1, 1), f32),
            pltpu.VMEM((1000, _NR), jnp.bfloat16),
            pltpu.VMEM((600, _NR), jnp.bfloat16),
            pltpu.VMEM((300, _NR), jnp.bfloat16),
            pltpu.VMEM((1, _NR), f32),
        ],
    )(xt, xr, *wg, *wl, *bgs)
    return jnp.concatenate([outs[0], outs[5]], axis=1).T


# speedup vs baseline: 1.5330x; 1.0360x over previous
import jax
import jax.numpy as jnp
from jax.experimental import pallas as pl
from jax.experimental.pallas import tpu as pltpu

_N = 20000
_TM = 1664
_STEPS = 12
_NM = _TM * _STEPS
_NR = _N - _NM


def _elu(v):
    return jnp.where(v > 0, v, jnp.exp(v) - 1.0)


def _mega_body(xt, xr, wg1, wg2, wg3, wg4, wl1, wl2, wl3, wl4,
               bg1, bg2, bg3, bg4,
               pred, u1, u2, u3, u4, predr,
               s0, s1, s2, s3, c1, c2, c3, c4, u1r, u2r, u3r, u4r):
    inv_n = 1.0 / _N
    bf = jnp.bfloat16
    f32 = jnp.float32

    for s in (s0, s1, s2, s3):
        s[...] = jnp.zeros_like(s)

    def col_specs(k, o, tm=_TM):
        return dict(in_specs=[pl.BlockSpec((k, tm), lambda i: (0, i))],
                    out_specs=[pl.BlockSpec((o, tm), lambda i: (0, i))])

    def head_body(x_v, u1_v):
        xb = x_v[...]
        u = jnp.dot(wg1[...], xb.astype(bf),
                    preferred_element_type=f32) + bg1[...]
        u1_v[...] = u.astype(bf)
        s0[...] += jnp.sum(xb, axis=1, keepdims=True)

    def mid_body(c_ref, wg, bg, s_ref):
        def body(uin_v, uout_v):
            h = _elu(uin_v[...].astype(f32) - c_ref[...])
            u = jnp.dot(wg[...], h.astype(bf),
                        preferred_element_type=f32) + bg[...]
            uout_v[...] = u.astype(uout_v.dtype)
            s_ref[...] += jnp.sum(h, axis=1, keepdims=True)
        return body

    def mean_col(wl, s_ref):
        return jnp.dot(wl[...], (s_ref[...] * inv_n).astype(bf),
                       preferred_element_type=f32)

    def rem_step(uin_r, c_ref, wg, bg, s_ref, uout_r):
        h = _elu(uin_r[...].astype(f32) - c_ref[...])
        s_ref[...] += jnp.sum(h, axis=1, keepdims=True)
        u = jnp.dot(wg[...], h.astype(bf),
                    preferred_element_type=f32) + bg[...]
        uout_r[...] = u.astype(uout_r.dtype)

    pltpu.emit_pipeline(head_body, grid=(_STEPS,),
                        **col_specs(1033, 1000))(xt, u1)
    xrb = xr[...]
    s0[...] += jnp.sum(xrb, axis=1, keepdims=True)
    u1r[...] = (jnp.dot(wg1[...], xrb.astype(bf),
                        preferred_element_type=f32) + bg1[...]).astype(bf)
    c1[...] = mean_col(wl1, s0)

    pltpu.emit_pipeline(mid_body(c1, wg2, bg2, s1), grid=(_NM // 3328,),
                        **col_specs(1000, 600, 3328))(u1, u2)
    rem_step(u1r, c1, wg2, bg2, s1, u2r)
    c2[...] = mean_col(wl2, s1)

    pltpu.emit_pipeline(mid_body(c2, wg3, bg3, s2), grid=(_NM // 3328,),
                        **col_specs(600, 300, 3328))(u2, u3)
    rem_step(u2r, c2, wg3, bg3, s2, u3r)
    c3[...] = mean_col(wl3, s2)

    pltpu.emit_pipeline(mid_body(c3, wg4, bg4, s3), grid=(_NM // 3328,),
                        **col_specs(300, 1, 3328))(u3, u4)
    rem_step(u3r, c3, wg4, bg4, s3, u4r)
    c4[...] = mean_col(wl4, s3)

    def tail_body(u4_v, pred_v):
        pred_v[...] = _elu(u4_v[...] - c4[...])

    pltpu.emit_pipeline(tail_body, grid=(_NM // 3328,),
                        **col_specs(1, 1, 3328))(u4, pred)
    predr[...] = _elu(u4r[...] - c4[...])


def kernel(x, Wg1, bg1, Wl1, Wg2, bg2, Wl2, Wg3, bg3, Wl3, Wg4, bg4, Wl4):
    bf = jnp.bfloat16
    f32 = jnp.float32
    wg = [w.astype(bf) for w in (Wg1, Wg2, Wg3, Wg4)]
    wl = [w.astype(bf) for w in (Wl1, Wl2, Wl3, Wl4)]
    bgs = [b.reshape(-1, 1).astype(f32) for b in (bg1, bg2, bg3, bg4)]

    xt = x.T
    xr = x[_NM:, :].T

    hbm = pl.BlockSpec(memory_space=pltpu.MemorySpace.HBM)
    vmem = pl.BlockSpec(memory_space=pltpu.MemorySpace.VMEM)

    outs = pl.pallas_call(
        _mega_body,
        in_specs=[hbm] + [vmem] * 13,
        out_specs=[hbm] * 5 + [vmem],
        out_shape=[
            jax.ShapeDtypeStruct((1, _NM), f32),
            jax.ShapeDtypeStruct((1000, _NM), bf),
            jax.ShapeDtypeStruct((600, _NM), bf),
            jax.ShapeDtypeStruct((300, _NM), bf),
            jax.ShapeDtypeStruct((1, _NM), f32),
            jax.ShapeDtypeStruct((1, _NR), f32),
        ],
        scratch_shapes=[
            pltpu.VMEM((1033, 1), f32),
            pltpu.VMEM((1000, 1), f32),
            pltpu.VMEM((600, 1), f32),
            pltpu.VMEM((300, 1), f32),
            pltpu.VMEM((1000, 1), f32),
            pltpu.VMEM((600, 1), f32),
            pltpu.VMEM((300, 1), f32),
            pltpu.VMEM((1, 1), f32),
            pltpu.VMEM((1000, _NR), jnp.bfloat16),
            pltpu.VMEM((600, _NR), jnp.bfloat16),
            pltpu.VMEM((300, _NR), jnp.bfloat16),
            pltpu.VMEM((1, _NR), f32),
        ],
    )(xt, xr, *wg, *wl, *bgs)
    return jnp.concatenate([outs[0], outs[5]], axis=1).T


# speedup vs baseline: 1.5480x; 1.0098x over previous
import jax
import jax.numpy as jnp
from jax.experimental import pallas as pl
from jax.experimental.pallas import tpu as pltpu

_N = 20000
_TM = 1664
_STEPS = 12
_NM = _TM * _STEPS
_NR = _N - _NM


def _elu(v):
    return jnp.where(v > 0, v, jnp.exp(v) - 1.0)


def _mega_body(xt, xr, wg1, wg2, wg3, wg4, wl1, wl2, wl3, wl4,
               bg1, bg2, bg3, bg4,
               pred, u1, u2, u3, u4, predr,
               s0, s1, s2, s3, c1, c2, c3, c4, u1r, u2r, u3r, u4r):
    inv_n = 1.0 / _N
    bf = jnp.bfloat16
    f32 = jnp.float32

    for s in (s0, s1, s2, s3):
        s[...] = jnp.zeros_like(s)

    def col_specs(k, o, tm=_TM):
        return dict(in_specs=[pl.BlockSpec((k, tm), lambda i: (0, i))],
                    out_specs=[pl.BlockSpec((o, tm), lambda i: (0, i))])

    def head_body(x_v, u1_v):
        xb = x_v[...]
        u = jnp.dot(wg1[...], xb.astype(bf),
                    preferred_element_type=f32) + bg1[...]
        u1_v[...] = u.astype(bf)
        s0[...] += jnp.sum(xb, axis=1, keepdims=True)

    def mid_body(c_ref, wg, bg, s_ref):
        def body(uin_v, uout_v):
            h = _elu(uin_v[...].astype(f32) - c_ref[...])
            u = jnp.dot(wg[...], h.astype(bf),
                        preferred_element_type=f32) + bg[...]
            uout_v[...] = u.astype(uout_v.dtype)
            s_ref[...] += jnp.sum(h, axis=1, keepdims=True)
        return body

    def mean_col(wl, s_ref):
        return jnp.dot(wl[...], (s_ref[...] * inv_n).astype(bf),
                       preferred_element_type=f32)

    def rem_step(uin_r, c_ref, wg, bg, s_ref, uout_r):
        h = _elu(uin_r[...].astype(f32) - c_ref[...])
        s_ref[...] += jnp.sum(h, axis=1, keepdims=True)
        u = jnp.dot(wg[...], h.astype(bf),
                    preferred_element_type=f32) + bg[...]
        uout_r[...] = u.astype(uout_r.dtype)

    pltpu.emit_pipeline(head_body, grid=(_NM // 3328,),
                        **col_specs(1033, 1000, 3328))(xt, u1)
    xrb = xr[...]
    s0[...] += jnp.sum(xrb, axis=1, keepdims=True)
    u1r[...] = (jnp.dot(wg1[...], xrb.astype(bf),
                        preferred_element_type=f32) + bg1[...]).astype(bf)
    c1[...] = mean_col(wl1, s0)

    pltpu.emit_pipeline(mid_body(c1, wg2, bg2, s1), grid=(_STEPS,),
                        **col_specs(1000, 600))(u1, u2)
    rem_step(u1r, c1, wg2, bg2, s1, u2r)
    c2[...] = mean_col(wl2, s1)

    pltpu.emit_pipeline(mid_body(c2, wg3, bg3, s2), grid=(_NM // 3328,),
                        **col_specs(600, 300, 3328))(u2, u3)
    rem_step(u2r, c2, wg3, bg3, s2, u3r)
    c3[...] = mean_col(wl3, s2)

    pltpu.emit_pipeline(mid_body(c3, wg4, bg4, s3), grid=(_NM // 3328,),
                        **col_specs(300, 1, 3328))(u3, u4)
    rem_step(u3r, c3, wg4, bg4, s3, u4r)
    c4[...] = mean_col(wl4, s3)

    def tail_body(u4_v, pred_v):
        pred_v[...] = _elu(u4_v[...] - c4[...])

    pltpu.emit_pipeline(tail_body, grid=(_NM // 3328,),
                        **col_specs(1, 1, 3328))(u4, pred)
    predr[...] = _elu(u4r[...] - c4[...])


def kernel(x, Wg1, bg1, Wl1, Wg2, bg2, Wl2, Wg3, bg3, Wl3, Wg4, bg4, Wl4):
    bf = jnp.bfloat16
    f32 = jnp.float32
    wg = [w.astype(bf) for w in (Wg1, Wg2, Wg3, Wg4)]
    wl = [w.astype(bf) for w in (Wl1, Wl2, Wl3, Wl4)]
    bgs = [b.reshape(-1, 1).astype(f32) for b in (bg1, bg2, bg3, bg4)]

    xt = x.T
    xr = x[_NM:, :].T

    hbm = pl.BlockSpec(memory_space=pltpu.MemorySpace.HBM)
    vmem = pl.BlockSpec(memory_space=pltpu.MemorySpace.VMEM)

    outs = pl.pallas_call(
        _mega_body,
        in_specs=[hbm] + [vmem] * 13,
        out_specs=[hbm] * 5 + [vmem],
        out_shape=[
            jax.ShapeDtypeStruct((1, _NM), f32),
            jax.ShapeDtypeStruct((1000, _NM), bf),
            jax.ShapeDtypeStruct((600, _NM), bf),
            jax.ShapeDtypeStruct((300, _NM), bf),
            jax.ShapeDtypeStruct((1, _NM), f32),
            jax.ShapeDtypeStruct((1, _NR), f32),
        ],
        scratch_shapes=[
            pltpu.VMEM((1033, 1), f32),
            pltpu.VMEM((1000, 1), f32),
            pltpu.VMEM((600, 1), f32),
            pltpu.VMEM((300, 1), f32),
            pltpu.VMEM((1000, 1), f32),
            pltpu.VMEM((600, 1), f32),
            pltpu.VMEM((300, 1), f32),
            pltpu.VMEM((1, 1), f32),
            pltpu.VMEM((1000, _NR), jnp.bfloat16),
            pltpu.VMEM((600, _NR), jnp.bfloat16),
            pltpu.VMEM((300, _NR), jnp.bfloat16),
            pltpu.VMEM((1, _NR), f32),
        ],
    )(xt, xr, *wg, *wl, *bgs)
    return jnp.concatenate([outs[0], outs[5]], axis=1).T
